# baseline (speedup 1.0000x reference)
import functools

import jax
import jax.numpy as jnp
from jax import lax
from jax.experimental import pallas as pl
from jax.experimental.pallas import tpu as pltpu
from jax.experimental.pallas import tpu_sc as plsc

_N_TOKENS = 8192
_N_CODES = 8192
_DIM = 256
_TBLK = 1024
_CBLK = 8192

_SC_NC = 2
_SC_NS = 16
_SC_BPW = _N_TOKENS // (_SC_NC * _SC_NS)


def _sc_gather(cb, idx):
    mesh = plsc.VectorSubcoreMesh(core_axis_name="c", subcore_axis_name="s")

    @functools.partial(
        pl.kernel, mesh=mesh,
        out_type=jax.ShapeDtypeStruct((_N_TOKENS, _DIM), jnp.float32),
        scratch_types=[
            pltpu.VMEM((_SC_BPW,), jnp.int32),
            pltpu.VMEM((_SC_BPW, _DIM), jnp.float32),
            pltpu.SemaphoreType.DMA,
        ],
    )
    def k(cb_hbm, idx_hbm, out_hbm, idx_v, rows_v, sem):
        wid = lax.axis_index("s") * _SC_NC + lax.axis_index("c")
        base = wid * _SC_BPW
        pltpu.sync_copy(idx_hbm.at[pl.ds(base, _SC_BPW)], idx_v)
        pltpu.async_copy(cb_hbm.at[idx_v], rows_v, sem).wait()
        pltpu.sync_copy(rows_v, out_hbm.at[pl.ds(base, _SC_BPW)])

    return k(cb, idx)


def _argmax_body(x_ref, cb_ref, t3h_ref, idx_ref, maxval, maxidx):
    c = pl.program_id(1)
    ncb = pl.num_programs(1)
    mm = jax.lax.dot_general(
        x_ref[...], cb_ref[...],
        dimension_numbers=(((1,), (1,)), ((), ())),
        preferred_element_type=jnp.float32,
    )
    s = mm - t3h_ref[...]
    bmax = jnp.max(s, axis=1, keepdims=True)
    iota = jax.lax.broadcasted_iota(jnp.int32, s.shape, 1)
    big = jnp.int32(2**30)
    bidx = jnp.min(jnp.where(s == bmax, iota, big), axis=1, keepdims=True)
    bidx = bidx + c * _CBLK

    @pl.when(c == 0)
    def _():
        maxval[...] = bmax
        maxidx[...] = bidx

    @pl.when(c > 0)
    def _():
        better = bmax > maxval[...]
        maxval[...] = jnp.where(better, bmax, maxval[...])
        maxidx[...] = jnp.where(better, bidx, maxidx[...])

    @pl.when(c == ncb - 1)
    def _():
        idx_ref[...] = maxidx[...]


def _stage_argmin(x, cb, t3h):
    grid = (_N_TOKENS // _TBLK, _N_CODES // _CBLK)
    return pl.pallas_call(
        _argmax_body,
        grid=grid,
        in_specs=[
            pl.BlockSpec((_TBLK, _DIM), lambda t, c: (t, 0)),
            pl.BlockSpec((_CBLK, _DIM), lambda t, c: (c, 0)),
            pl.BlockSpec((1, _CBLK), lambda t, c: (0, c)),
        ],
        out_specs=pl.BlockSpec((_TBLK, 1), lambda t, c: (t, 0)),
        out_shape=jax.ShapeDtypeStruct((_N_TOKENS, 1), jnp.int32),
        scratch_shapes=[
            pltpu.VMEM((_TBLK, 1), jnp.float32),
            pltpu.VMEM((_TBLK, 1), jnp.int32),
        ],
    )(x, cb, t3h)


def kernel(input_data, codebooks):
    remainder = input_data
    final_quantized = jnp.zeros_like(input_data)
    used = []
    for i in range(codebooks.shape[0]):
        cb = codebooks[i]
        t3h = 0.5 * jnp.sum(cb.T ** 2, axis=0, keepdims=True)
        min_idx = _stage_argmin(remainder, cb, t3h)[:, 0]
        q = _sc_gather(cb, min_idx)
        remainder = remainder - q
        final_quantized = final_quantized + q
        used.append(
            jnp.zeros((_N_CODES,), jnp.int32).at[min_idx].set(1))
    codebooks_used = jnp.stack(used, axis=0)
    return final_quantized, codebooks_used, codebooks


# speedup vs baseline: 1.1053x; 1.1053x over previous
import functools

import jax
import jax.numpy as jnp
from jax import lax
from jax.experimental import pallas as pl
from jax.experimental.pallas import tpu as pltpu
from jax.experimental.pallas import tpu_sc as plsc

_N_TOKENS = 8192
_N_CODES = 8192
_DIM = 256
_TBLK = 1024
_CBLK = 8192

_SC_NC = 2
_SC_NS = 16
_SC_BPW = _N_TOKENS // (_SC_NC * _SC_NS)


def _sc_gather(cb, idx):
    mesh = plsc.VectorSubcoreMesh(core_axis_name="c", subcore_axis_name="s")

    @functools.partial(
        pl.kernel, mesh=mesh,
        out_type=jax.ShapeDtypeStruct((_N_TOKENS, _DIM), jnp.float32),
        scratch_types=[
            pltpu.VMEM((_SC_BPW,), jnp.int32),
            pltpu.VMEM((_SC_BPW, _DIM), jnp.float32),
            pltpu.SemaphoreType.DMA,
        ],
    )
    def k(cb_hbm, idx_hbm, out_hbm, idx_v, rows_v, sem):
        wid = lax.axis_index("s") * _SC_NC + lax.axis_index("c")
        base = wid * _SC_BPW
        pltpu.sync_copy(idx_hbm.at[pl.ds(base, _SC_BPW)], idx_v)
        pltpu.async_copy(cb_hbm.at[idx_v], rows_v, sem).wait()
        pltpu.sync_copy(rows_v, out_hbm.at[pl.ds(base, _SC_BPW)])

    return k(cb, idx)


def _argmax_body(x_ref, cb_ref, t3h_ref, idx_ref, maxval, maxidx):
    c = pl.program_id(1)
    ncb = pl.num_programs(1)
    mm = jax.lax.dot_general(
        x_ref[...], cb_ref[...],
        dimension_numbers=(((1,), (1,)), ((), ())),
        preferred_element_type=jnp.float32,
    )
    s = mm - t3h_ref[...]
    bmax = jnp.max(s, axis=1, keepdims=True)
    bidx = jnp.argmax(s, axis=1).astype(jnp.int32).reshape(s.shape[0], 1)
    bidx = bidx + c * _CBLK

    @pl.when(c == 0)
    def _():
        maxval[...] = bmax
        maxidx[...] = bidx

    @pl.when(c > 0)
    def _():
        better = bmax > maxval[...]
        maxval[...] = jnp.where(better, bmax, maxval[...])
        maxidx[...] = jnp.where(better, bidx, maxidx[...])

    @pl.when(c == ncb - 1)
    def _():
        idx_ref[...] = maxidx[...]


def _stage_argmin(x, cb, t3h):
    grid = (_N_TOKENS // _TBLK, _N_CODES // _CBLK)
    return pl.pallas_call(
        _argmax_body,
        grid=grid,
        in_specs=[
            pl.BlockSpec((_TBLK, _DIM), lambda t, c: (t, 0)),
            pl.BlockSpec((_CBLK, _DIM), lambda t, c: (c, 0)),
            pl.BlockSpec((1, _CBLK), lambda t, c: (0, c)),
        ],
        out_specs=pl.BlockSpec((_TBLK, 1), lambda t, c: (t, 0)),
        out_shape=jax.ShapeDtypeStruct((_N_TOKENS, 1), jnp.int32),
        scratch_shapes=[
            pltpu.VMEM((_TBLK, 1), jnp.float32),
            pltpu.VMEM((_TBLK, 1), jnp.int32),
        ],
    )(x, cb, t3h)


def kernel(input_data, codebooks):
    remainder = input_data
    final_quantized = jnp.zeros_like(input_data)
    used = []
    for i in range(codebooks.shape[0]):
        cb = codebooks[i]
        t3h = 0.5 * jnp.sum(cb.T ** 2, axis=0, keepdims=True)
        min_idx = _stage_argmin(remainder, cb, t3h)[:, 0]
        q = _sc_gather(cb, min_idx)
        remainder = remainder - q
        final_quantized = final_quantized + q
        used.append(
            jnp.zeros((_N_CODES,), jnp.int32).at[min_idx].set(1))
    codebooks_used = jnp.stack(used, axis=0)
    return final_quantized, codebooks_used, codebooks


# speedup vs baseline: 1.3231x; 1.1970x over previous
import functools

import jax
import jax.numpy as jnp
from jax import lax
from jax.experimental import pallas as pl
from jax.experimental.pallas import tpu as pltpu
from jax.experimental.pallas import tpu_sc as plsc

_N_TOKENS = 8192
_N_CODES = 8192
_DIM = 256
_TBLK = 1024

_SC_NC = 2
_SC_NS = 16
_SC_BPW = _N_TOKENS // (_SC_NC * _SC_NS)


def _sc_gather(cb, idx):
    mesh = plsc.VectorSubcoreMesh(core_axis_name="c", subcore_axis_name="s")

    @functools.partial(
        pl.kernel, mesh=mesh,
        out_type=jax.ShapeDtypeStruct((_N_TOKENS, _DIM), jnp.float32),
        scratch_types=[
            pltpu.VMEM((_SC_BPW,), jnp.int32),
            pltpu.VMEM((_SC_BPW, _DIM), jnp.float32),
            pltpu.SemaphoreType.DMA,
        ],
    )
    def k(cb_hbm, idx_hbm, out_hbm, idx_v, rows_v, sem):
        wid = lax.axis_index("s") * _SC_NC + lax.axis_index("c")
        base = wid * _SC_BPW
        pltpu.sync_copy(idx_hbm.at[pl.ds(base, _SC_BPW)], idx_v)
        pltpu.async_copy(cb_hbm.at[idx_v], rows_v, sem).wait()
        pltpu.sync_copy(rows_v, out_hbm.at[pl.ds(base, _SC_BPW)])

    return k(cb, idx)


def _argmax_body(x_ref, cb_ref, t3h_ref, idx_ref):
    mm = jax.lax.dot_general(
        x_ref[...], cb_ref[...],
        dimension_numbers=(((1,), (1,)), ((), ())),
        preferred_element_type=jnp.float32,
    )
    s = mm - t3h_ref[...]
    idx_ref[...] = (
        jnp.argmax(s, axis=1).astype(jnp.int32).reshape(s.shape[0], 1))


def _stage_argmin(x, cb, t3h):
    return pl.pallas_call(
        _argmax_body,
        grid=(_N_TOKENS // _TBLK,),
        in_specs=[
            pl.BlockSpec((_TBLK, _DIM), lambda t: (t, 0)),
            pl.BlockSpec((_N_CODES, _DIM), lambda t: (0, 0)),
            pl.BlockSpec((1, _N_CODES), lambda t: (0, 0)),
        ],
        out_specs=pl.BlockSpec((_TBLK, 1), lambda t: (t, 0)),
        out_shape=jax.ShapeDtypeStruct((_N_TOKENS, 1), jnp.int32),
    )(x, cb, t3h)


def kernel(input_data, codebooks):
    remainder = input_data
    final_quantized = jnp.zeros_like(input_data)
    used = []
    for i in range(codebooks.shape[0]):
        cb = codebooks[i]
        t3h = 0.5 * jnp.sum(cb.T ** 2, axis=0, keepdims=True)
        min_idx = _stage_argmin(remainder, cb, t3h)[:, 0]
        q = _sc_gather(cb, min_idx)
        remainder = remainder - q
        final_quantized = final_quantized + q
        used.append(
            jnp.zeros((_N_CODES,), jnp.int32).at[min_idx].set(1))
    codebooks_used = jnp.stack(used, axis=0)
    return final_quantized, codebooks_used, codebooks
